# Initial kernel scaffold; baseline (speedup 1.0000x reference)
import jax
import jax.numpy as jnp
from jax.experimental import pallas as pl


def kernel(user_ids, item_ids, edge_index, table, W1, b1, W2, b2, Wp, bp):
    raise NotImplementedError("write your pallas kernel here")


# speedup vs baseline: 30.5864x; 30.5864x over previous
import functools

import jax
import jax.numpy as jnp
from jax import lax
from jax.experimental import pallas as pl
from jax.experimental.pallas import tpu as pltpu
from jax.experimental.pallas import tpu_sc as plsc

NU = 25000
NI = 25000
N = NU + NI
D = 32
NC, NS = 2, 16
NW = NC * NS

E = 1_600_000
EPAD = 1_605_632
EROWS = EPAD // 128
ER_TEC = EROWS // NW
EBLK = 4
ENB = ER_TEC // EBLK

NPAD = 50176
AGG_TEC = NPAD // NS

IDPAD = 53248
IDROWS = IDPAD // 128
ID_TEC = IDROWS // NW
HB = IDPAD
HB_TEC = HB // NS

OPAD = 28672
O_TEC = OPAD // NW


def _mesh():
    return plsc.VectorSubcoreMesh(core_axis_name="c", subcore_axis_name="s")


def _a_body(ids3, dst3, table, x_out, degp, idsv, dstv, xbuf, hist, sem):
    cid = lax.axis_index("c")
    sid = lax.axis_index("s")
    wid = cid * NS + sid

    def _zh(i, c):
        hist[pl.ds(i * 16, 16)] = jnp.zeros((16,), jnp.float32)
        return c
    lax.fori_loop(0, HB // 16, _zh, 0)

    pltpu.sync_copy(ids3.at[pl.ds(wid * ID_TEC, ID_TEC), :, :], idsv)
    for j in range(ID_TEC):
        pltpu.async_copy(table.at[idsv.at[j, 0]], xbuf, sem).wait()
        pltpu.sync_copy(xbuf, x_out.at[pl.ds((wid * ID_TEC + j) * 128, 128), :])

    ccnt, _ = plsc.scan_count(jnp.zeros((16,), jnp.int32))
    base = (16 - jnp.max(ccnt)).astype(jnp.float32)

    def _blk(b, c):
        pltpu.sync_copy(
            dst3.at[pl.ds(wid * ER_TEC + b * EBLK, EBLK), :, :], dstv)
        for r in range(EBLK):
            for g in range(8):
                d16 = dstv[r, 0, pl.ds(g * 16, 16)]
                cnt, lst = plsc.scan_count(d16)
                cur = plsc.load_gather(hist, [d16], mask=lst)
                plsc.store_scatter(
                    hist, [d16], cur + cnt.astype(jnp.float32) + base, mask=lst)
        return c
    lax.fori_loop(0, ENB, _blk, 0)

    pltpu.sync_copy(hist, degp.at[pl.ds(wid * HB, HB)])


def _call_a(ids3, dst3, table):
    f = pl.kernel(
        _a_body,
        out_type=[
            jax.ShapeDtypeStruct((IDPAD, D), jnp.float32),
            jax.ShapeDtypeStruct((NW * HB,), jnp.float32),
        ],
        mesh=_mesh(),
        compiler_params=pltpu.CompilerParams(needs_layout_passes=False, use_tc_tiling_on_sc=False),
        scratch_types=[
            pltpu.VMEM((ID_TEC, 1, 128), jnp.int32),
            pltpu.VMEM((EBLK, 1, 128), jnp.int32),
            pltpu.VMEM((128, D), jnp.float32),
            pltpu.VMEM((HB,), jnp.float32),
            pltpu.SemaphoreType.DMA,
        ],
        name="gcn_lookup_deg",
    )
    return f(ids3, dst3, table)


_NTEC = ID_TEC * 128


def _a2_body(degp, dinv32, accv, rowv, obuf):
    cid = lax.axis_index("c")
    sid = lax.axis_index("s")
    wid = cid * NS + sid
    nb0 = wid * _NTEC

    def _za(i, c):
        accv[pl.ds(i * 16, 16)] = jnp.full((16,), 1.0, jnp.float32)
        return c
    lax.fori_loop(0, _NTEC // 16, _za, 0)

    def _part(p, c):
        pltpu.sync_copy(degp.at[pl.ds(p * HB + nb0, _NTEC)], rowv)

        def _acc(i, cc):
            accv[pl.ds(i * 16, 16)] = (
                accv[pl.ds(i * 16, 16)] + rowv[pl.ds(i * 16, 16)])
            return cc
        lax.fori_loop(0, _NTEC // 16, _acc, 0)
        return c
    lax.fori_loop(0, NW, _part, 0)

    def _chunk(j, c):
        for g in range(8):
            d = accv[pl.ds(j * 128 + g * 16, 16)]
            ii = plsc.bitcast(d, jnp.int32)
            ii = 0x5F3759DF - jnp.right_shift(ii, 1)
            y = plsc.bitcast(ii, jnp.float32)
            h = 0.5 * d
            y = y * (1.5 - h * y * y)
            y = y * (1.5 - h * y * y)
            y = y * (1.5 - h * y * y)
            for l in range(16):
                v = jnp.full((16,), y[l], dtype=jnp.float32)
                obuf[g * 16 + l, pl.ds(0, 16)] = v
                obuf[g * 16 + l, pl.ds(16, 16)] = v
        pltpu.sync_copy(obuf, dinv32.at[pl.ds(nb0 + j * 128, 128), :])
        return c

    lax.fori_loop(0, ID_TEC, _chunk, 0)


def _call_a2(degp):
    f = pl.kernel(
        _a2_body,
        out_type=jax.ShapeDtypeStruct((IDPAD, D), jnp.float32),
        mesh=_mesh(),
        compiler_params=pltpu.CompilerParams(needs_layout_passes=False, use_tc_tiling_on_sc=False),
        scratch_types=[
            pltpu.VMEM((_NTEC,), jnp.float32),
            pltpu.VMEM((_NTEC,), jnp.float32),
            pltpu.VMEM((128, D), jnp.float32),
        ],
        name="gcn_dinv",
    )
    return f(degp)


def _spmm_body(y, src3, dst3, outp, srcv, dstv, rows, zbuf, agg, gsem, ssem):
    cid = lax.axis_index("c")
    sid = lax.axis_index("s")
    wid = cid * NS + sid

    def _zr(i, c):
        zbuf[i, pl.ds(0, 16)] = jnp.zeros((16,), jnp.float32)
        zbuf[i, pl.ds(16, 16)] = jnp.zeros((16,), jnp.float32)
        return c
    lax.fori_loop(0, 64, _zr, 0)

    def _zc(i, c):
        pltpu.sync_copy(zbuf, agg.at[pl.ds(sid * AGG_TEC + i * 64, 64), :])
        return c
    lax.fori_loop(0, AGG_TEC // 64, _zc, 0)
    plsc.subcore_barrier()

    def _blk(b, c):
        rb = wid * ER_TEC + b * EBLK
        pltpu.sync_copy(src3.at[pl.ds(rb, EBLK), :, :], srcv)
        pltpu.sync_copy(dst3.at[pl.ds(rb, EBLK), :, :], dstv)
        for j in range(EBLK):
            pltpu.async_copy(
                y.at[srcv.at[j, 0]], rows.at[pl.ds(j * 128, 128), :], gsem)
        for j in range(EBLK):
            pltpu.make_async_copy(
                y.at[srcv.at[j, 0]], rows.at[pl.ds(j * 128, 128), :],
                gsem).wait()
        for j in range(EBLK):
            pltpu.async_copy(
                rows.at[pl.ds(j * 128, 128), :], agg.at[dstv.at[j, 0]], ssem,
                add=True)
        for j in range(EBLK):
            pltpu.make_async_copy(
                rows.at[pl.ds(j * 128, 128), :], agg.at[dstv.at[j, 0]],
                ssem).wait()
        return c
    lax.fori_loop(0, ENB, _blk, 0)
    plsc.subcore_barrier()
    pltpu.sync_copy(
        agg.at[pl.ds(sid * AGG_TEC, AGG_TEC), :],
        outp.at[pl.ds(cid * NPAD + sid * AGG_TEC, AGG_TEC), :])


def _call_spmm(y, src3, dst3):
    f = pl.kernel(
        _spmm_body,
        out_type=jax.ShapeDtypeStruct((NC * NPAD, D), jnp.float32),
        mesh=_mesh(),
        compiler_params=pltpu.CompilerParams(needs_layout_passes=False, use_tc_tiling_on_sc=False),
        scratch_types=[
            pltpu.VMEM((EBLK, 1, 128), jnp.int32),
            pltpu.VMEM((EBLK, 1, 128), jnp.int32),
            pltpu.VMEM((EBLK * 128, D), jnp.float32),
            pltpu.VMEM((64, D), jnp.float32),
            pltpu.VMEM_SHARED((NPAD, D), jnp.float32),
            pltpu.SemaphoreType.DMA,
            pltpu.SemaphoreType.DMA,
        ],
        name="gcn_spmm",
    )
    return f(y, src3, dst3)


def _g_body(h2, uid, iid, wp, bpv, outo, uv, iv, rU, rI, wv, bv, ov, gsem):
    cid = lax.axis_index("c")
    sid = lax.axis_index("s")
    wid = cid * NS + sid
    pltpu.sync_copy(wp, wv)
    pltpu.sync_copy(bpv, bv)
    wp0a = wv[pl.ds(0, 16)]
    wp0b = wv[pl.ds(16, 16)]
    wp1a = wv[pl.ds(32, 16)]
    wp1b = wv[pl.ds(48, 16)]
    bps = bv[pl.ds(0, 16)][0]
    lane = lax.iota(jnp.int32, 16)

    def _chunk(ch, c):
        base = wid * O_TEC + ch * 128
        pltpu.sync_copy(uid.at[pl.ds(base, 128)], uv)
        pltpu.sync_copy(iid.at[pl.ds(base, 128)], iv)
        for g in range(8):
            iv[pl.ds(g * 16, 16)] = iv[pl.ds(g * 16, 16)] + NU
        pltpu.async_copy(h2.at[uv], rU, gsem)
        pltpu.async_copy(h2.at[iv], rI, gsem)
        pltpu.make_async_copy(h2.at[uv], rU, gsem).wait()
        pltpu.make_async_copy(h2.at[iv], rI, gsem).wait()
        for g in range(8):
            out16 = jnp.full((16,), 0.0, dtype=jnp.float32)
            for l in range(16):
                j = g * 16 + l
                t = (rU[j, pl.ds(0, 16)] * wp0a + rU[j, pl.ds(16, 16)] * wp0b
                     + rI[j, pl.ds(0, 16)] * wp1a
                     + rI[j, pl.ds(16, 16)] * wp1b)
                s = jnp.sum(t) + bps
                out16 = jnp.where(lane == l, s, out16)
            ov[pl.ds(g * 16, 16)] = out16
        pltpu.sync_copy(ov, outo.at[pl.ds(base, 128)])
        return c

    lax.fori_loop(0, O_TEC // 128, _chunk, 0)


def _call_g(h2, uid, iid, wp, bpv):
    f = pl.kernel(
        _g_body,
        out_type=jax.ShapeDtypeStruct((OPAD,), jnp.float32),
        mesh=_mesh(),
        compiler_params=pltpu.CompilerParams(needs_layout_passes=False, use_tc_tiling_on_sc=False),
        scratch_types=[
            pltpu.VMEM((128,), jnp.int32),
            pltpu.VMEM((128,), jnp.int32),
            pltpu.VMEM((128, D), jnp.float32),
            pltpu.VMEM((128, D), jnp.float32),
            pltpu.VMEM((64,), jnp.float32),
            pltpu.VMEM((16,), jnp.float32),
            pltpu.VMEM((128,), jnp.float32),
            pltpu.SemaphoreType.DMA,
        ],
        name="gcn_head",
    )
    return f(h2, uid, iid, wp, bpv)


_BLK = 1792
_NBLK = NPAD // _BLK


def _tc_b_body(x_r, dv_r, w_r, y_r):
    y_r[...] = jnp.dot(x_r[...] * dv_r[...], w_r[...],
                       preferred_element_type=jnp.float32)


def _tc_d_body(p0_r, p1_r, y_r, dv_r, b_r, w_r, o_r):
    dv = dv_r[...]
    h = jnp.maximum((p0_r[...] + p1_r[...] + y_r[...]) * dv + b_r[...], 0.0)
    o_r[...] = jnp.dot(h * dv, w_r[...], preferred_element_type=jnp.float32)


def _tc_f_body(p0_r, p1_r, y_r, dv_r, b_r, o_r):
    o_r[...] = jnp.maximum(
        (p0_r[...] + p1_r[...] + y_r[...]) * dv_r[...] + b_r[...], 0.0)


_row_spec = pl.BlockSpec((_BLK, D), lambda i: (i, 0))
_w_spec = pl.BlockSpec((D, D), lambda i: (0, 0))
_b_spec = pl.BlockSpec((1, D), lambda i: (0, 0))


def _call_tc_b(x, dinv32, W1):
    return pl.pallas_call(
        _tc_b_body,
        grid=(_NBLK,),
        in_specs=[_row_spec, _row_spec, _w_spec],
        out_specs=_row_spec,
        out_shape=jax.ShapeDtypeStruct((NPAD, D), jnp.float32),
    )(x, dinv32, W1)


def _call_tc_d(aggp, y1, dinv32, b1, W2):
    p1_spec = pl.BlockSpec((_BLK, D), lambda i: (i + _NBLK, 0))
    return pl.pallas_call(
        _tc_d_body,
        grid=(_NBLK,),
        in_specs=[_row_spec, p1_spec, _row_spec, _row_spec, _b_spec, _w_spec],
        out_specs=_row_spec,
        out_shape=jax.ShapeDtypeStruct((NPAD, D), jnp.float32),
    )(aggp, aggp, y1, dinv32, b1, W2)


def _call_tc_f(aggp, y2, dinv32, b2):
    p1_spec = pl.BlockSpec((_BLK, D), lambda i: (i + _NBLK, 0))
    return pl.pallas_call(
        _tc_f_body,
        grid=(_NBLK,),
        in_specs=[_row_spec, p1_spec, _row_spec, _row_spec, _b_spec],
        out_specs=_row_spec,
        out_shape=jax.ShapeDtypeStruct((NPAD, D), jnp.float32),
    )(aggp, aggp, y2, dinv32, b2)


def kernel(user_ids, item_ids, edge_index, table, W1, b1, W2, b2, Wp, bp):
    i32 = jnp.int32
    user_ids = user_ids.astype(i32)
    item_ids = item_ids.astype(i32)

    pad_ids = jnp.arange(IDPAD - N, dtype=i32) % NU
    ids3 = jnp.concatenate([user_ids, item_ids, pad_ids]).reshape(
        IDROWS, 1, 128)

    npe = EPAD - E
    src3 = jnp.concatenate(
        [edge_index[0].astype(i32), jnp.arange(npe, dtype=i32) % N]
    ).reshape(EROWS, 1, 128)
    dst3 = jnp.concatenate(
        [edge_index[1].astype(i32), N + (jnp.arange(npe, dtype=i32) % (NPAD - N))]
    ).reshape(EROWS, 1, 128)

    npo = OPAD - NU
    uid = jnp.concatenate([user_ids, jnp.arange(npo, dtype=i32) % NU])
    iid = jnp.concatenate([item_ids, jnp.arange(npo, dtype=i32) % NI])

    wp_flat = Wp.reshape(2 * D)
    bp_pad = jnp.concatenate([bp, jnp.zeros((15,), jnp.float32)])
    b1_2d = b1.reshape(1, D)
    b2_2d = b2.reshape(1, D)

    x, degp = _call_a(ids3, dst3, table)
    dinv32 = _call_a2(degp)
    y1 = _call_tc_b(x[:NPAD], dinv32[:NPAD], W1)
    aggp1 = _call_spmm(y1, src3, dst3)
    y2 = _call_tc_d(aggp1, y1, dinv32[:NPAD], b1_2d, W2)
    aggp2 = _call_spmm(y2, src3, dst3)
    h2 = _call_tc_f(aggp2, y2, dinv32[:NPAD], b2_2d)
    out = _call_g(h2, uid, iid, wp_flat, bp_pad)
    return out[:NU].reshape(NU, 1)


# speedup vs baseline: 45.9964x; 1.5038x over previous
import functools

import jax
import jax.numpy as jnp
from jax import lax
from jax.experimental import pallas as pl
from jax.experimental.pallas import tpu as pltpu
from jax.experimental.pallas import tpu_sc as plsc

NU = 25000
NI = 25000
N = NU + NI
D = 32
NC, NS = 2, 16
NW = NC * NS

E = 1_600_000
EPAD = 1_622_016
EROWS = EPAD // 128
ER_TEC = EROWS // NW
EBLK = 4
ENB = ER_TEC // EBLK
RING = 6
RITER = ER_TEC // RING

NPAD = 50176
AGG_TEC = NPAD // NS

IDPAD = 53248
IDROWS = IDPAD // 128
ID_TEC = IDROWS // NW
HB = IDPAD
HB_TEC = HB // NS

OPAD = 28672
O_TEC = OPAD // NW


def _mesh():
    return plsc.VectorSubcoreMesh(core_axis_name="c", subcore_axis_name="s")


def _a_body(ids3, dst3, table, x_out, degp, idsv, dstv, xbuf, hist, sem):
    cid = lax.axis_index("c")
    sid = lax.axis_index("s")
    wid = cid * NS + sid

    def _zh(i, c):
        hist[pl.ds(i * 16, 16)] = jnp.zeros((16,), jnp.float32)
        return c
    lax.fori_loop(0, HB // 16, _zh, 0)

    pltpu.sync_copy(ids3.at[pl.ds(wid * ID_TEC, ID_TEC), :, :], idsv)
    for j in range(ID_TEC):
        pltpu.async_copy(table.at[idsv.at[j, 0]], xbuf, sem).wait()
        pltpu.sync_copy(xbuf, x_out.at[pl.ds((wid * ID_TEC + j) * 128, 128), :])

    ccnt, _ = plsc.scan_count(jnp.zeros((16,), jnp.int32))
    base = (16 - jnp.max(ccnt)).astype(jnp.float32)

    def _blk(b, c):
        pltpu.sync_copy(
            dst3.at[pl.ds(wid * ER_TEC + b * EBLK, EBLK), :, :], dstv)
        for r in range(EBLK):
            for g in range(8):
                d16 = dstv[r, 0, pl.ds(g * 16, 16)]
                cnt, lst = plsc.scan_count(d16)
                cur = plsc.load_gather(hist, [d16], mask=lst)
                plsc.store_scatter(
                    hist, [d16], cur + cnt.astype(jnp.float32) + base, mask=lst)
        return c
    lax.fori_loop(0, ENB, _blk, 0)

    pltpu.sync_copy(hist, degp.at[pl.ds(wid * HB, HB)])


def _call_a(ids3, dst3, table):
    f = pl.kernel(
        _a_body,
        out_type=[
            jax.ShapeDtypeStruct((IDPAD, D), jnp.float32),
            jax.ShapeDtypeStruct((NW * HB,), jnp.float32),
        ],
        mesh=_mesh(),
        compiler_params=pltpu.CompilerParams(needs_layout_passes=False, use_tc_tiling_on_sc=False),
        scratch_types=[
            pltpu.VMEM((ID_TEC, 1, 128), jnp.int32),
            pltpu.VMEM((EBLK, 1, 128), jnp.int32),
            pltpu.VMEM((128, D), jnp.float32),
            pltpu.VMEM((HB,), jnp.float32),
            pltpu.SemaphoreType.DMA,
        ],
        name="gcn_lookup_deg",
    )
    return f(ids3, dst3, table)


_NTEC = ID_TEC * 128


def _a2_body(degp, dinv32, accv, rowv, obuf):
    cid = lax.axis_index("c")
    sid = lax.axis_index("s")
    wid = cid * NS + sid
    nb0 = wid * _NTEC

    def _za(i, c):
        accv[pl.ds(i * 16, 16)] = jnp.full((16,), 1.0, jnp.float32)
        return c
    lax.fori_loop(0, _NTEC // 16, _za, 0)

    def _part(p, c):
        pltpu.sync_copy(degp.at[pl.ds(p * HB + nb0, _NTEC)], rowv)

        def _acc(i, cc):
            accv[pl.ds(i * 16, 16)] = (
                accv[pl.ds(i * 16, 16)] + rowv[pl.ds(i * 16, 16)])
            return cc
        lax.fori_loop(0, _NTEC // 16, _acc, 0)
        return c
    lax.fori_loop(0, NW, _part, 0)

    def _chunk(j, c):
        for g in range(8):
            d = accv[pl.ds(j * 128 + g * 16, 16)]
            ii = plsc.bitcast(d, jnp.int32)
            ii = 0x5F3759DF - jnp.right_shift(ii, 1)
            y = plsc.bitcast(ii, jnp.float32)
            h = 0.5 * d
            y = y * (1.5 - h * y * y)
            y = y * (1.5 - h * y * y)
            y = y * (1.5 - h * y * y)
            for l in range(16):
                v = jnp.full((16,), y[l], dtype=jnp.float32)
                obuf[g * 16 + l, pl.ds(0, 16)] = v
                obuf[g * 16 + l, pl.ds(16, 16)] = v
        pltpu.sync_copy(obuf, dinv32.at[pl.ds(nb0 + j * 128, 128), :])
        return c

    lax.fori_loop(0, ID_TEC, _chunk, 0)


def _call_a2(degp):
    f = pl.kernel(
        _a2_body,
        out_type=jax.ShapeDtypeStruct((IDPAD, D), jnp.float32),
        mesh=_mesh(),
        compiler_params=pltpu.CompilerParams(needs_layout_passes=False, use_tc_tiling_on_sc=False),
        scratch_types=[
            pltpu.VMEM((_NTEC,), jnp.float32),
            pltpu.VMEM((_NTEC,), jnp.float32),
            pltpu.VMEM((128, D), jnp.float32),
        ],
        name="gcn_dinv",
    )
    return f(degp)


def _spmm_body(y, src3, dst3, outp, srcA, dstA, srcB, dstB, rows, agg,
               gsems, ssem, isem):
    cid = lax.axis_index("c")
    sid = lax.axis_index("s")
    wid = cid * NS + sid
    rb0 = wid * ER_TEC

    def _zr(i, c):
        rows[i, pl.ds(0, 16)] = jnp.zeros((16,), jnp.float32)
        rows[i, pl.ds(16, 16)] = jnp.zeros((16,), jnp.float32)
        return c
    lax.fori_loop(0, 64, _zr, 0)

    def _zc(i, c):
        pltpu.sync_copy(
            rows.at[pl.ds(0, 64), :],
            agg.at[pl.ds(sid * AGG_TEC + i * 64, 64), :])
        return c
    lax.fori_loop(0, AGG_TEC // 64, _zc, 0)
    plsc.subcore_barrier()

    def _slot(j):
        return rows.at[pl.ds(j * 128, 128), :]

    def _fire_gathers(sv):
        for j in range(RING):
            pltpu.async_copy(y.at[sv.at[j, 0]], _slot(j), gsems.at[j])

    def _gather_wait_scatter(sv, dv):
        for j in range(RING):
            pltpu.make_async_copy(
                y.at[sv.at[j, 0]], _slot(j), gsems.at[j]).wait()
            pltpu.async_copy(_slot(j), agg.at[dv.at[j, 0]], ssem, add=True)

    def _drain_scatters(dv):
        for j in range(RING):
            pltpu.make_async_copy(_slot(j), agg.at[dv.at[j, 0]], ssem).wait()

    def _fill_idx(i, sv, dv):
        rb = rb0 + i * RING
        pltpu.async_copy(src3.at[pl.ds(rb, RING), :, :], sv, isem)
        pltpu.async_copy(dst3.at[pl.ds(rb, RING), :, :], dv, isem)

    def _wait_idx(i, sv, dv):
        rb = rb0 + i * RING
        pltpu.make_async_copy(src3.at[pl.ds(rb, RING), :, :], sv, isem).wait()
        pltpu.make_async_copy(dst3.at[pl.ds(rb, RING), :, :], dv, isem).wait()

    pltpu.sync_copy(src3.at[pl.ds(rb0, RING), :, :], srcA)
    pltpu.sync_copy(dst3.at[pl.ds(rb0, RING), :, :], dstA)
    _fire_gathers(srcA)
    _fill_idx(1, srcB, dstB)
    _gather_wait_scatter(srcA, dstA)

    def _pair(k, c):
        i = 1 + 2 * k
        _wait_idx(i, srcB, dstB)
        _drain_scatters(dstA)
        _fire_gathers(srcB)
        _fill_idx(i + 1, srcA, dstA)
        _gather_wait_scatter(srcB, dstB)
        _wait_idx(i + 1, srcA, dstA)
        _drain_scatters(dstB)
        _fire_gathers(srcA)
        _fill_idx(i + 2, srcB, dstB)
        _gather_wait_scatter(srcA, dstA)
        return c
    lax.fori_loop(0, (RITER - 2) // 2, _pair, 0)

    _wait_idx(RITER - 1, srcB, dstB)
    _drain_scatters(dstA)
    _fire_gathers(srcB)
    _gather_wait_scatter(srcB, dstB)
    _drain_scatters(dstB)

    plsc.subcore_barrier()
    pltpu.sync_copy(
        agg.at[pl.ds(sid * AGG_TEC, AGG_TEC), :],
        outp.at[pl.ds(cid * NPAD + sid * AGG_TEC, AGG_TEC), :])


def _call_spmm(y, src3, dst3):
    f = pl.kernel(
        _spmm_body,
        out_type=jax.ShapeDtypeStruct((NC * NPAD, D), jnp.float32),
        mesh=_mesh(),
        compiler_params=pltpu.CompilerParams(needs_layout_passes=False, use_tc_tiling_on_sc=False),
        scratch_types=[
            pltpu.VMEM((RING, 1, 128), jnp.int32),
            pltpu.VMEM((RING, 1, 128), jnp.int32),
            pltpu.VMEM((RING, 1, 128), jnp.int32),
            pltpu.VMEM((RING, 1, 128), jnp.int32),
            pltpu.VMEM((RING * 128, D), jnp.float32),
            pltpu.VMEM_SHARED((NPAD, D), jnp.float32),
            pltpu.SemaphoreType.DMA((RING,)),
            pltpu.SemaphoreType.DMA,
            pltpu.SemaphoreType.DMA,
        ],
        name="gcn_spmm",
    )
    return f(y, src3, dst3)


def _g_body(h2, uid, iid, wp, bpv, outo, uv, iv, rU, rI, wv, bv, ov, gsem):
    cid = lax.axis_index("c")
    sid = lax.axis_index("s")
    wid = cid * NS + sid
    pltpu.sync_copy(wp, wv)
    pltpu.sync_copy(bpv, bv)
    wp0a = wv[pl.ds(0, 16)]
    wp0b = wv[pl.ds(16, 16)]
    wp1a = wv[pl.ds(32, 16)]
    wp1b = wv[pl.ds(48, 16)]
    bps = bv[pl.ds(0, 16)][0]
    lane = lax.iota(jnp.int32, 16)

    def _chunk(ch, c):
        base = wid * O_TEC + ch * 128
        pltpu.sync_copy(uid.at[pl.ds(base, 128)], uv)
        pltpu.sync_copy(iid.at[pl.ds(base, 128)], iv)
        for g in range(8):
            iv[pl.ds(g * 16, 16)] = iv[pl.ds(g * 16, 16)] + NU
        pltpu.async_copy(h2.at[uv], rU, gsem)
        pltpu.async_copy(h2.at[iv], rI, gsem)
        pltpu.make_async_copy(h2.at[uv], rU, gsem).wait()
        pltpu.make_async_copy(h2.at[iv], rI, gsem).wait()
        for g in range(8):
            out16 = jnp.full((16,), 0.0, dtype=jnp.float32)
            for l in range(16):
                j = g * 16 + l
                t = (rU[j, pl.ds(0, 16)] * wp0a + rU[j, pl.ds(16, 16)] * wp0b
                     + rI[j, pl.ds(0, 16)] * wp1a
                     + rI[j, pl.ds(16, 16)] * wp1b)
                s = jnp.sum(t) + bps
                out16 = jnp.where(lane == l, s, out16)
            ov[pl.ds(g * 16, 16)] = out16
        pltpu.sync_copy(ov, outo.at[pl.ds(base, 128)])
        return c

    lax.fori_loop(0, O_TEC // 128, _chunk, 0)


def _call_g(h2, uid, iid, wp, bpv):
    f = pl.kernel(
        _g_body,
        out_type=jax.ShapeDtypeStruct((OPAD,), jnp.float32),
        mesh=_mesh(),
        compiler_params=pltpu.CompilerParams(needs_layout_passes=False, use_tc_tiling_on_sc=False),
        scratch_types=[
            pltpu.VMEM((128,), jnp.int32),
            pltpu.VMEM((128,), jnp.int32),
            pltpu.VMEM((128, D), jnp.float32),
            pltpu.VMEM((128, D), jnp.float32),
            pltpu.VMEM((64,), jnp.float32),
            pltpu.VMEM((16,), jnp.float32),
            pltpu.VMEM((128,), jnp.float32),
            pltpu.SemaphoreType.DMA,
        ],
        name="gcn_head",
    )
    return f(h2, uid, iid, wp, bpv)


M4 = NPAD // 4
IDM4 = IDPAD // 4
_BLKR = 448
_NBLK = M4 // _BLKR


def _tc_b_body(x_r, dv_r, w_r, y_r):
    y_r[...] = jnp.dot(x_r[...] * dv_r[...], w_r[...],
                       preferred_element_type=jnp.float32)


def _tc_d_body(p0_r, p1_r, y_r, dv_r, b_r, w_r, o_r):
    dv = dv_r[...]
    h = jnp.maximum((p0_r[...] + p1_r[...] + y_r[...]) * dv + b_r[...], 0.0)
    o_r[...] = jnp.dot(h * dv, w_r[...], preferred_element_type=jnp.float32)


def _tc_f_body(p0_r, p1_r, y_r, dv_r, b_r, o_r):
    o_r[...] = jnp.maximum(
        (p0_r[...] + p1_r[...] + y_r[...]) * dv_r[...] + b_r[...], 0.0)


_row_spec = pl.BlockSpec((_BLKR, 128), lambda i: (i, 0))
_w_spec = pl.BlockSpec((128, 128), lambda i: (0, 0))
_b_spec = pl.BlockSpec((1, 128), lambda i: (0, 0))


def _call_tc_b(x4, dinv4, W1bd):
    return pl.pallas_call(
        _tc_b_body,
        grid=(_NBLK,),
        in_specs=[_row_spec, _row_spec, _w_spec],
        out_specs=_row_spec,
        out_shape=jax.ShapeDtypeStruct((M4, 128), jnp.float32),
    )(x4, dinv4, W1bd)


def _call_tc_d(aggp4, y1_4, dinv4, b1r, W2bd):
    p1_spec = pl.BlockSpec((_BLKR, 128), lambda i: (i + _NBLK, 0))
    return pl.pallas_call(
        _tc_d_body,
        grid=(_NBLK,),
        in_specs=[_row_spec, p1_spec, _row_spec, _row_spec, _b_spec, _w_spec],
        out_specs=_row_spec,
        out_shape=jax.ShapeDtypeStruct((M4, 128), jnp.float32),
    )(aggp4, aggp4, y1_4, dinv4, b1r, W2bd)


def _call_tc_f(aggp4, y2_4, dinv4, b2r):
    p1_spec = pl.BlockSpec((_BLKR, 128), lambda i: (i + _NBLK, 0))
    return pl.pallas_call(
        _tc_f_body,
        grid=(_NBLK,),
        in_specs=[_row_spec, p1_spec, _row_spec, _row_spec, _b_spec],
        out_specs=_row_spec,
        out_shape=jax.ShapeDtypeStruct((M4, 128), jnp.float32),
    )(aggp4, aggp4, y2_4, dinv4, b2r)


def kernel(user_ids, item_ids, edge_index, table, W1, b1, W2, b2, Wp, bp):
    i32 = jnp.int32
    user_ids = user_ids.astype(i32)
    item_ids = item_ids.astype(i32)

    pad_ids = jnp.arange(IDPAD - N, dtype=i32) % NU
    ids3 = jnp.concatenate([user_ids, item_ids, pad_ids]).reshape(
        IDROWS, 1, 128)

    npe = EPAD - E
    src3 = jnp.concatenate(
        [edge_index[0].astype(i32), jnp.arange(npe, dtype=i32) % N]
    ).reshape(EROWS, 1, 128)
    dst3 = jnp.concatenate(
        [edge_index[1].astype(i32), N + (jnp.arange(npe, dtype=i32) % (NPAD - N))]
    ).reshape(EROWS, 1, 128)

    npo = OPAD - NU
    uid = jnp.concatenate([user_ids, jnp.arange(npo, dtype=i32) % NU])
    iid = jnp.concatenate([item_ids, jnp.arange(npo, dtype=i32) % NI])

    wp_flat = Wp.reshape(2 * D)
    bp_pad = jnp.concatenate([bp, jnp.zeros((15,), jnp.float32)])
    eye4 = jnp.eye(4, dtype=jnp.float32)
    W1bd = jnp.kron(eye4, W1)
    W2bd = jnp.kron(eye4, W2)
    b1r = jnp.tile(b1, 4).reshape(1, 128)
    b2r = jnp.tile(b2, 4).reshape(1, 128)

    x, degp = _call_a(ids3, dst3, table)
    dinv32 = _call_a2(degp)
    x4 = x.reshape(IDM4, 128)[:M4]
    dinv4 = dinv32.reshape(IDM4, 128)[:M4]
    y1_4 = _call_tc_b(x4, dinv4, W1bd)
    y1 = y1_4.reshape(NPAD, D)
    aggp1 = _call_spmm(y1, src3, dst3)
    y2_4 = _call_tc_d(aggp1.reshape(2 * M4, 128), y1_4, dinv4, b1r, W2bd)
    y2 = y2_4.reshape(NPAD, D)
    aggp2 = _call_spmm(y2, src3, dst3)
    h2_4 = _call_tc_f(aggp2.reshape(2 * M4, 128), y2_4, dinv4, b2r)
    h2 = h2_4.reshape(NPAD, D)
    out = _call_g(h2, uid, iid, wp_flat, bp_pad)
    return out[:NU].reshape(NU, 1)


# speedup vs baseline: 47.6474x; 1.0359x over previous
import jax
import jax.numpy as jnp
from jax import lax
from jax.experimental import pallas as pl
from jax.experimental.pallas import tpu as pltpu
from jax.experimental.pallas import tpu_sc as plsc

NU = 25000
NI = 25000
N = NU + NI
D = 32
NC, NS = 2, 16
NW = NC * NS

E = 1_600_000
EPAD = 1_622_016
ECW = 256
ECROWS = EPAD // ECW
EC_TEC = ECROWS // NW
RING = 3
RITER = EC_TEC // RING
AEB = 2
ANB = EC_TEC // AEB

NPAD = 50176
AGG_TEC = NPAD // NS

IDPAD = 53248
IDROWS = IDPAD // 128
ID_TEC = IDROWS // NW
HB = IDPAD

OPAD = 28672
O_TEC = OPAD // NW

_SC_PARAMS = dict(
    compiler_params=pltpu.CompilerParams(
        needs_layout_passes=False, use_tc_tiling_on_sc=False))


def _mesh():
    return plsc.VectorSubcoreMesh(core_axis_name="c", subcore_axis_name="s")


def _a_body(ids3, dst3, table, x_out, degp, idsv, dstv, dstv2, xbuf, hist,
            hist2, gsems, wsem, isem):
    cid = lax.axis_index("c")
    sid = lax.axis_index("s")
    wid = cid * NS + sid
    rb0 = wid * EC_TEC

    def _zh(i, c):
        z = jnp.zeros((16,), jnp.float32)
        hist[pl.ds(i * 16, 16)] = z
        hist2[pl.ds(i * 16, 16)] = z
        return c
    lax.fori_loop(0, HB // 16, _zh, 0)

    pltpu.sync_copy(ids3.at[pl.ds(wid * ID_TEC, ID_TEC), :, :], idsv)

    def _xslot(j):
        return xbuf.at[pl.ds((j % 4) * 128, 128), :]

    for j in range(4):
        pltpu.async_copy(table.at[idsv.at[j, 0]], _xslot(j), gsems.at[j % 4])
    for j in range(ID_TEC):
        pltpu.make_async_copy(
            table.at[idsv.at[j, 0]], _xslot(j), gsems.at[j % 4]).wait()
        pltpu.async_copy(
            _xslot(j), x_out.at[pl.ds((wid * ID_TEC + j) * 128, 128), :],
            wsem)
        if j + 4 < ID_TEC:
            pltpu.make_async_copy(
                _xslot(j), x_out.at[pl.ds((wid * ID_TEC + j) * 128, 128), :],
                wsem).wait()
            pltpu.async_copy(
                table.at[idsv.at[j + 4, 0]], _xslot(j + 4),
                gsems.at[j % 4])

    ccnt, _ = plsc.scan_count(jnp.zeros((16,), jnp.int32))
    base = (16 - jnp.max(ccnt)).astype(jnp.float32)

    def _grp(dv):
        for r in range(AEB):
            for g in range(ECW // 16):
                hh = hist if g % 2 == 0 else hist2
                d16 = dv[r, 0, pl.ds(g * 16, 16)]
                cnt, lst = plsc.scan_count(d16)
                cur = plsc.load_gather(hh, [d16], mask=lst)
                plsc.store_scatter(
                    hh, [d16], cur + cnt.astype(jnp.float32) + base, mask=lst)

    def _fill(b, dv):
        pltpu.async_copy(dst3.at[pl.ds(rb0 + b * AEB, AEB), :, :], dv, isem)

    def _wait_fill(b, dv):
        pltpu.make_async_copy(
            dst3.at[pl.ds(rb0 + b * AEB, AEB), :, :], dv, isem).wait()

    pltpu.sync_copy(dst3.at[pl.ds(rb0, AEB), :, :], dstv)
    _fill(1, dstv2)
    _grp(dstv)

    def _pairblk(k, c):
        b = 1 + 2 * k
        _wait_fill(b, dstv2)
        _fill(b + 1, dstv)
        _grp(dstv2)
        _wait_fill(b + 1, dstv)

        @pl.when(k < (ANB - 1) // 2 - 1)
        def _():
            _fill(b + 2, dstv2)
        _grp(dstv)
        return c
    lax.fori_loop(0, (ANB - 1) // 2, _pairblk, 0)

    def _mh(i, c):
        hist[pl.ds(i * 16, 16)] = (
            hist[pl.ds(i * 16, 16)] + hist2[pl.ds(i * 16, 16)])
        return c
    lax.fori_loop(0, HB // 16, _mh, 0)

    for j in range(ID_TEC - 4, ID_TEC):
        pltpu.make_async_copy(
            _xslot(j), x_out.at[pl.ds((wid * ID_TEC + j) * 128, 128), :],
            wsem).wait()
    pltpu.sync_copy(hist, degp.at[pl.ds(wid * HB, HB)])


def _call_a(ids3, dst3, table):
    f = pl.kernel(
        _a_body,
        out_type=[
            jax.ShapeDtypeStruct((IDPAD, D), jnp.float32),
            jax.ShapeDtypeStruct((NW * HB,), jnp.float32),
        ],
        mesh=_mesh(),
        scratch_types=[
            pltpu.VMEM((ID_TEC, 1, 128), jnp.int32),
            pltpu.VMEM((AEB, 1, ECW), jnp.int32),
            pltpu.VMEM((AEB, 1, ECW), jnp.int32),
            pltpu.VMEM((512, D), jnp.float32),
            pltpu.VMEM((HB,), jnp.float32),
            pltpu.VMEM((HB,), jnp.float32),
            pltpu.SemaphoreType.DMA((4,)),
            pltpu.SemaphoreType.DMA,
            pltpu.SemaphoreType.DMA,
        ],
        name="gcn_lookup_deg",
        **_SC_PARAMS,
    )
    return f(ids3, dst3, table)


_NTEC = ID_TEC * 128


def _a2_body(degp, dinv32, accv, rowv, obuf, psems):
    cid = lax.axis_index("c")
    sid = lax.axis_index("s")
    wid = cid * NS + sid
    nb0 = wid * _NTEC

    def _za(i, c):
        accv[pl.ds(i * 16, 16)] = jnp.full((16,), 1.0, jnp.float32)
        return c
    lax.fori_loop(0, _NTEC // 16, _za, 0)

    for q in range(4):
        pltpu.async_copy(
            degp.at[pl.ds(q * HB + nb0, _NTEC)], rowv.at[q], psems.at[q])

    def _p4(k, c):
        for q in range(4):
            p = 4 * k + q
            pltpu.make_async_copy(
                degp.at[pl.ds(p * HB + nb0, _NTEC)], rowv.at[q],
                psems.at[q]).wait()

            def _acc(i, cc):
                accv[pl.ds(i * 16, 16)] = (
                    accv[pl.ds(i * 16, 16)] + rowv[q, pl.ds(i * 16, 16)])
                return cc
            lax.fori_loop(0, _NTEC // 16, _acc, 0)

            @pl.when(p + 4 < NW)
            def _():
                pltpu.async_copy(
                    degp.at[pl.ds((p + 4) * HB + nb0, _NTEC)], rowv.at[q],
                    psems.at[q])
        return c
    lax.fori_loop(0, NW // 4, _p4, 0)

    def _chunk(j, c):
        for g in range(8):
            d = accv[pl.ds(j * 128 + g * 16, 16)]
            ii = plsc.bitcast(d, jnp.int32)
            ii = 0x5F3759DF - jnp.right_shift(ii, 1)
            y = plsc.bitcast(ii, jnp.float32)
            h = 0.5 * d
            y = y * (1.5 - h * y * y)
            y = y * (1.5 - h * y * y)
            y = y * (1.5 - h * y * y)
            for l in range(16):
                v = jnp.full((16,), y[l], dtype=jnp.float32)
                obuf[g * 16 + l, pl.ds(0, 16)] = v
                obuf[g * 16 + l, pl.ds(16, 16)] = v
        pltpu.sync_copy(obuf, dinv32.at[pl.ds(nb0 + j * 128, 128), :])
        return c

    lax.fori_loop(0, ID_TEC, _chunk, 0)


def _call_a2(degp):
    f = pl.kernel(
        _a2_body,
        out_type=jax.ShapeDtypeStruct((IDPAD, D), jnp.float32),
        mesh=_mesh(),
        scratch_types=[
            pltpu.VMEM((_NTEC,), jnp.float32),
            pltpu.VMEM((4, _NTEC), jnp.float32),
            pltpu.VMEM((128, D), jnp.float32),
            pltpu.SemaphoreType.DMA((4,)),
        ],
        name="gcn_dinv",
        **_SC_PARAMS,
    )
    return f(degp)


def _spmm_body(y, src3, dst3, outp, srcA, dstA, srcB, dstB, rows, agg,
               gsems, ssem, isem):
    cid = lax.axis_index("c")
    sid = lax.axis_index("s")
    wid = cid * NS + sid
    rb0 = wid * EC_TEC

    def _zr(i, c):
        rows[i, pl.ds(0, 16)] = jnp.zeros((16,), jnp.float32)
        rows[i, pl.ds(16, 16)] = jnp.zeros((16,), jnp.float32)
        return c
    lax.fori_loop(0, 64, _zr, 0)

    def _zc(i, c):
        pltpu.sync_copy(
            rows.at[pl.ds(0, 64), :],
            agg.at[pl.ds(sid * AGG_TEC + i * 64, 64), :])
        return c
    lax.fori_loop(0, AGG_TEC // 64, _zc, 0)
    plsc.subcore_barrier()

    def _slot(j):
        return rows.at[pl.ds(j * ECW, ECW), :]

    def _fire_gathers(sv):
        for j in range(RING):
            pltpu.async_copy(y.at[sv.at[j, 0]], _slot(j), gsems.at[j])

    def _gather_wait_scatter(sv, dv):
        for j in range(RING):
            pltpu.make_async_copy(
                y.at[sv.at[j, 0]], _slot(j), gsems.at[j]).wait()
            pltpu.async_copy(_slot(j), agg.at[dv.at[j, 0]], ssem, add=True)

    def _drain_scatters(dv):
        for j in range(RING):
            pltpu.make_async_copy(_slot(j), agg.at[dv.at[j, 0]], ssem).wait()

    def _fill_idx(i, sv, dv):
        rb = rb0 + i * RING
        pltpu.async_copy(src3.at[pl.ds(rb, RING), :, :], sv, isem)
        pltpu.async_copy(dst3.at[pl.ds(rb, RING), :, :], dv, isem)

    def _wait_idx(i, sv, dv):
        rb = rb0 + i * RING
        pltpu.make_async_copy(src3.at[pl.ds(rb, RING), :, :], sv, isem).wait()
        pltpu.make_async_copy(dst3.at[pl.ds(rb, RING), :, :], dv, isem).wait()

    pltpu.sync_copy(src3.at[pl.ds(rb0, RING), :, :], srcA)
    pltpu.sync_copy(dst3.at[pl.ds(rb0, RING), :, :], dstA)
    _fire_gathers(srcA)
    _fill_idx(1, srcB, dstB)
    _gather_wait_scatter(srcA, dstA)

    def _pair(k, c):
        i = 1 + 2 * k
        _wait_idx(i, srcB, dstB)
        _drain_scatters(dstA)
        _fire_gathers(srcB)
        _fill_idx(i + 1, srcA, dstA)
        _gather_wait_scatter(srcB, dstB)
        _wait_idx(i + 1, srcA, dstA)
        _drain_scatters(dstB)
        _fire_gathers(srcA)
        _fill_idx(i + 2, srcB, dstB)
        _gather_wait_scatter(srcA, dstA)
        return c
    lax.fori_loop(0, (RITER - 2) // 2, _pair, 0)

    _wait_idx(RITER - 1, srcB, dstB)
    _drain_scatters(dstA)
    _fire_gathers(srcB)
    _gather_wait_scatter(srcB, dstB)
    _drain_scatters(dstB)

    plsc.subcore_barrier()
    pltpu.sync_copy(
        agg.at[pl.ds(sid * AGG_TEC, AGG_TEC), :],
        outp.at[pl.ds(cid * NPAD + sid * AGG_TEC, AGG_TEC), :])


def _call_spmm(y, src3, dst3):
    f = pl.kernel(
        _spmm_body,
        out_type=jax.ShapeDtypeStruct((NC * NPAD, D), jnp.float32),
        mesh=_mesh(),
        scratch_types=[
            pltpu.VMEM((RING, 1, ECW), jnp.int32),
            pltpu.VMEM((RING, 1, ECW), jnp.int32),
            pltpu.VMEM((RING, 1, ECW), jnp.int32),
            pltpu.VMEM((RING, 1, ECW), jnp.int32),
            pltpu.VMEM((RING * ECW, D), jnp.float32),
            pltpu.VMEM_SHARED((NPAD, D), jnp.float32),
            pltpu.SemaphoreType.DMA((RING,)),
            pltpu.SemaphoreType.DMA,
            pltpu.SemaphoreType.DMA,
        ],
        name="gcn_spmm",
        **_SC_PARAMS,
    )
    return f(y, src3, dst3)


def _g_body(h2, uid, iid, wp, bpv, outo, uv, iv, rU, rI, wv, bv, ov, gsem):
    cid = lax.axis_index("c")
    sid = lax.axis_index("s")
    wid = cid * NS + sid
    pltpu.sync_copy(wp, wv)
    pltpu.sync_copy(bpv, bv)
    wp0a = wv[pl.ds(0, 16)]
    wp0b = wv[pl.ds(16, 16)]
    wp1a = wv[pl.ds(32, 16)]
    wp1b = wv[pl.ds(48, 16)]
    bps = bv[pl.ds(0, 16)][0]
    lane = lax.iota(jnp.int32, 16)

    def _chunk(ch, c):
        base = wid * O_TEC + ch * 128
        pltpu.sync_copy(uid.at[pl.ds(base, 128)], uv)
        pltpu.sync_copy(iid.at[pl.ds(base, 128)], iv)
        for g in range(8):
            iv[pl.ds(g * 16, 16)] = iv[pl.ds(g * 16, 16)] + NU
        pltpu.async_copy(h2.at[uv], rU, gsem)
        pltpu.async_copy(h2.at[iv], rI, gsem)
        pltpu.make_async_copy(h2.at[uv], rU, gsem).wait()
        pltpu.make_async_copy(h2.at[iv], rI, gsem).wait()
        for g in range(8):
            out16 = jnp.full((16,), 0.0, dtype=jnp.float32)
            for l in range(16):
                j = g * 16 + l
                t = (rU[j, pl.ds(0, 16)] * wp0a + rU[j, pl.ds(16, 16)] * wp0b
                     + rI[j, pl.ds(0, 16)] * wp1a
                     + rI[j, pl.ds(16, 16)] * wp1b)
                s = jnp.sum(t) + bps
                out16 = jnp.where(lane == l, s, out16)
            ov[pl.ds(g * 16, 16)] = out16
        pltpu.sync_copy(ov, outo.at[pl.ds(base, 128)])
        return c

    lax.fori_loop(0, O_TEC // 128, _chunk, 0)


def _call_g(h2, uid, iid, wp, bpv):
    f = pl.kernel(
        _g_body,
        out_type=jax.ShapeDtypeStruct((OPAD,), jnp.float32),
        mesh=_mesh(),
        scratch_types=[
            pltpu.VMEM((128,), jnp.int32),
            pltpu.VMEM((128,), jnp.int32),
            pltpu.VMEM((128, D), jnp.float32),
            pltpu.VMEM((128, D), jnp.float32),
            pltpu.VMEM((64,), jnp.float32),
            pltpu.VMEM((16,), jnp.float32),
            pltpu.VMEM((128,), jnp.float32),
            pltpu.SemaphoreType.DMA,
        ],
        name="gcn_head",
        **_SC_PARAMS,
    )
    return f(h2, uid, iid, wp, bpv)


M4 = NPAD // 4
IDM4 = IDPAD // 4
_BLKR = 448
_NBLK = M4 // _BLKR


def _tc_b_body(x_r, dv_r, w_r, y_r):
    y_r[...] = jnp.dot(x_r[...] * dv_r[...], w_r[...],
                       preferred_element_type=jnp.float32)


def _tc_d_body(p0_r, p1_r, y_r, dv_r, b_r, w_r, o_r):
    dv = dv_r[...]
    h = jnp.maximum((p0_r[...] + p1_r[...] + y_r[...]) * dv + b_r[...], 0.0)
    o_r[...] = jnp.dot(h * dv, w_r[...], preferred_element_type=jnp.float32)


def _tc_f_body(p0_r, p1_r, y_r, dv_r, b_r, o_r):
    o_r[...] = jnp.maximum(
        (p0_r[...] + p1_r[...] + y_r[...]) * dv_r[...] + b_r[...], 0.0)


_row_spec = pl.BlockSpec((_BLKR, 128), lambda i: (i, 0))
_w_spec = pl.BlockSpec((128, 128), lambda i: (0, 0))
_b_spec = pl.BlockSpec((1, 128), lambda i: (0, 0))


def _call_tc_b(x4, dinv4, W1bd):
    return pl.pallas_call(
        _tc_b_body,
        grid=(_NBLK,),
        in_specs=[_row_spec, _row_spec, _w_spec],
        out_specs=_row_spec,
        out_shape=jax.ShapeDtypeStruct((M4, 128), jnp.float32),
    )(x4, dinv4, W1bd)


def _call_tc_d(aggp4, y1_4, dinv4, b1r, W2bd):
    p1_spec = pl.BlockSpec((_BLKR, 128), lambda i: (i + _NBLK, 0))
    return pl.pallas_call(
        _tc_d_body,
        grid=(_NBLK,),
        in_specs=[_row_spec, p1_spec, _row_spec, _row_spec, _b_spec, _w_spec],
        out_specs=_row_spec,
        out_shape=jax.ShapeDtypeStruct((M4, 128), jnp.float32),
    )(aggp4, aggp4, y1_4, dinv4, b1r, W2bd)


def _call_tc_f(aggp4, y2_4, dinv4, b2r):
    p1_spec = pl.BlockSpec((_BLKR, 128), lambda i: (i + _NBLK, 0))
    return pl.pallas_call(
        _tc_f_body,
        grid=(_NBLK,),
        in_specs=[_row_spec, p1_spec, _row_spec, _row_spec, _b_spec],
        out_specs=_row_spec,
        out_shape=jax.ShapeDtypeStruct((M4, 128), jnp.float32),
    )(aggp4, aggp4, y2_4, dinv4, b2r)


def kernel(user_ids, item_ids, edge_index, table, W1, b1, W2, b2, Wp, bp):
    i32 = jnp.int32
    user_ids = user_ids.astype(i32)
    item_ids = item_ids.astype(i32)

    pad_ids = jnp.arange(IDPAD - N, dtype=i32) % NU
    ids3 = jnp.concatenate([user_ids, item_ids, pad_ids]).reshape(
        IDROWS, 1, 128)

    npe = EPAD - E
    src3 = jnp.concatenate(
        [edge_index[0].astype(i32), jnp.arange(npe, dtype=i32) % N]
    ).reshape(ECROWS, 1, ECW)
    dst3 = jnp.concatenate(
        [edge_index[1].astype(i32), N + (jnp.arange(npe, dtype=i32) % (NPAD - N))]
    ).reshape(ECROWS, 1, ECW)

    npo = OPAD - NU
    uid = jnp.concatenate([user_ids, jnp.arange(npo, dtype=i32) % NU])
    iid = jnp.concatenate([item_ids, jnp.arange(npo, dtype=i32) % NI])

    wp_flat = Wp.reshape(2 * D)
    bp_pad = jnp.concatenate([bp, jnp.zeros((15,), jnp.float32)])
    eye4 = jnp.eye(4, dtype=jnp.float32)
    W1bd = jnp.kron(eye4, W1)
    W2bd = jnp.kron(eye4, W2)
    b1r = jnp.tile(b1, 4).reshape(1, 128)
    b2r = jnp.tile(b2, 4).reshape(1, 128)

    x, degp = _call_a(ids3, dst3, table)
    dinv32 = _call_a2(degp)
    x4 = x.reshape(IDM4, 128)[:M4]
    dinv4 = dinv32.reshape(IDM4, 128)[:M4]
    y1_4 = _call_tc_b(x4, dinv4, W1bd)
    y1 = y1_4.reshape(NPAD, D)
    aggp1 = _call_spmm(y1, src3, dst3)
    y2_4 = _call_tc_d(aggp1.reshape(2 * M4, 128), y1_4, dinv4, b1r, W2bd)
    y2 = y2_4.reshape(NPAD, D)
    aggp2 = _call_spmm(y2, src3, dst3)
    h2_4 = _call_tc_f(aggp2.reshape(2 * M4, 128), y2_4, dinv4, b2r)
    h2 = h2_4.reshape(NPAD, D)
    out = _call_g(h2, uid, iid, wp_flat, bp_pad)
    return out[:NU].reshape(NU, 1)
